# baseline (speedup 1.0000x reference)
import functools

import jax
import jax.numpy as jnp
from jax import lax
from jax.experimental import pallas as pl
from jax.experimental.pallas import tpu as pltpu
from jax.experimental.pallas import tpu_sc as plsc

N = 10000
E = 320000
D = 128
H1 = 64
H2 = 32
OUT = 32

CH = 125
NW = 32
ROWS_PER_TILE = E // (NW * CH)
NPAD = 10240
STRIPE = NPAD // 16

_HIGH = jax.lax.Precision.HIGHEST


def _make_sc_agg(H, with_deg):
    mesh = plsc.VectorSubcoreMesh(core_axis_name="c", subcore_axis_name="s")
    out_type = [jax.ShapeDtypeStruct((2, NPAD, H), jnp.float32)]
    if with_deg:
        out_type.append(jax.ShapeDtypeStruct((NPAD,), jnp.float32))
        out_type.append(jax.ShapeDtypeStruct((NPAD,), jnp.float32))
    scratch = [
        pltpu.VMEM((ROWS_PER_TILE, CH), jnp.int32),
        pltpu.VMEM((ROWS_PER_TILE, CH), jnp.int32),
    ]
    nb = 5 if H > 32 else 8
    scratch += [pltpu.VMEM((CH, H), jnp.float32)] * nb
    scratch += [
        pltpu.VMEM((128, H), jnp.float32),
        pltpu.VMEM_SHARED((NPAD, H), jnp.float32),
    ] + [pltpu.SemaphoreType.DMA] * nb
    if with_deg:
        scratch += [
            pltpu.VMEM((CH,), jnp.float32),
            pltpu.VMEM((STRIPE,), jnp.float32),
            pltpu.VMEM_SHARED((NPAD,), jnp.float32),
        ]

    @functools.partial(pl.kernel, mesh=mesh, out_type=tuple(out_type),
                       scratch_types=scratch,
                       compiler_params=pltpu.CompilerParams(
                           use_tc_tiling_on_sc=False))
    def k(xl, src2, dst2, *refs):
        if with_deg:
            (agg_out, deg0_out, deg1_out, src_v, dst_v, *rest) = refs
        else:
            (agg_out, src_v, dst_v, *rest) = refs
        nb = 5 if H > 32 else 8
        bufs = tuple(rest[0:nb])
        zblk = rest[nb]
        acc = rest[nb + 1]
        sems = tuple(rest[nb + 2:2 * nb + 2])
        if with_deg:
            ones_v, zvec, dacc = rest[2 * nb + 2:2 * nb + 5]
        c = lax.axis_index("c")
        s = lax.axis_index("s")
        wid = s * 2 + c
        zero16 = jnp.zeros((16,), jnp.float32)

        def zfill(i, _):
            zblk[i // (H // 16), pl.ds((i % (H // 16)) * 16, 16)] = zero16
            return 0
        lax.fori_loop(0, 128 * (H // 16), zfill, 0)

        def zacc(t, _):
            pltpu.sync_copy(zblk, acc.at[pl.ds(s * STRIPE + t * 128, 128), :])
            return 0
        lax.fori_loop(0, STRIPE // 128, zacc, 0)

        if with_deg:
            def zdfill(i, _):
                zvec[pl.ds(i * 16, 16)] = zero16
                return 0
            lax.fori_loop(0, STRIPE // 16, zdfill, 0)
            pltpu.sync_copy(zvec, dacc.at[pl.ds(s * STRIPE, STRIPE)])
            one16 = jnp.ones((16,), jnp.float32)

            def ofill(i, _):
                ones_v[pl.ds(jnp.minimum(i * 16, CH - 16), 16)] = one16
                return 0
            lax.fori_loop(0, (CH + 15) // 16, ofill, 0)

        plsc.subcore_barrier()

        row0 = wid * ROWS_PER_TILE
        pltpu.sync_copy(src2.at[pl.ds(row0, ROWS_PER_TILE), :], src_v)
        pltpu.sync_copy(dst2.at[pl.ds(row0, ROWS_PER_TILE), :], dst_v)

        NB = nb

        def drain_g(b, sm):
            pltpu.make_async_copy(xl.at[pl.ds(0, CH), :], b, sm).wait()

        for b in range(NB):
            pltpu.async_copy(xl.at[src_v.at[b]], bufs[b], sems[b])

        def step(i, _):
            for b in range(NB):
                j = NB * i + b
                drain_g(bufs[b], sems[b])
                pltpu.sync_copy(bufs[b], acc.at[dst_v.at[j]], add=True)
                if with_deg:
                    pltpu.sync_copy(ones_v, dacc.at[dst_v.at[j]], add=True)
                jn = jnp.minimum(j + NB, ROWS_PER_TILE - 1)
                pltpu.async_copy(xl.at[src_v.at[jn]], bufs[b], sems[b])
            return 0
        lax.fori_loop(0, ROWS_PER_TILE // NB, step, 0)
        for b in range(NB):
            drain_g(bufs[b], sems[b])

        plsc.subcore_barrier()

        pltpu.sync_copy(acc.at[pl.ds(s * STRIPE, STRIPE), :],
                        agg_out.at[c].at[pl.ds(s * STRIPE, STRIPE), :])
        if with_deg:
            @pl.when(c == 0)
            def _():
                pltpu.sync_copy(dacc.at[pl.ds(s * STRIPE, STRIPE)],
                                deg0_out.at[pl.ds(s * STRIPE, STRIPE)])

            @pl.when(c == 1)
            def _():
                pltpu.sync_copy(dacc.at[pl.ds(s * STRIPE, STRIPE)],
                                deg1_out.at[pl.ds(s * STRIPE, STRIPE)])

    return k


_sc_agg64 = _make_sc_agg(H1, with_deg=True)
_sc_agg32 = _make_sc_agg(H2, with_deg=False)


_BR = 5000


def _mm2(x, Wl, Wr):
    Din, H = Wl.shape

    def body(x_ref, wl_ref, wr_ref, ol_ref, or_ref):
        xb = x_ref[...]
        ol_ref[...] = jnp.dot(xb, wl_ref[...], precision=_HIGH,
                              preferred_element_type=jnp.float32)
        or_ref[...] = jnp.dot(xb, wr_ref[...], precision=_HIGH,
                              preferred_element_type=jnp.float32)

    return pl.pallas_call(
        body,
        grid=(N // _BR,),
        in_specs=[pl.BlockSpec((_BR, Din), lambda i: (i, 0)),
                  pl.BlockSpec((Din, H), lambda i: (0, 0)),
                  pl.BlockSpec((Din, H), lambda i: (0, 0))],
        out_specs=[pl.BlockSpec((_BR, H), lambda i: (i, 0)),
                   pl.BlockSpec((_BR, H), lambda i: (i, 0))],
        out_shape=[jax.ShapeDtypeStruct((N, H), jnp.float32)] * 2,
    )(x, Wl, Wr)


def _layer_out(a0, a1, inv, xr, b):
    out = (a0 + a1) * inv + b + xr
    nrm = jnp.sqrt(jnp.sum(out * out, axis=1, keepdims=True))
    return out / jnp.maximum(nrm, 1e-12)


def _combine1(agg, d0, d1, xr, b, Wl, Wr):
    H, Hn = Wl.shape

    def body(a0_r, a1_r, d0_r, d1_r, xr_r, b_r, wl_r, wr_r,
             ol_r, or_r, oi_r):
        inv = 1.0 / jnp.maximum(d0_r[...] + d1_r[...], 1.0)
        h = _layer_out(a0_r[0], a1_r[0], inv, xr_r[...], b_r[...])
        h = jnp.maximum(h, 0.0)
        ol_r[...] = jnp.dot(h, wl_r[...], precision=_HIGH,
                            preferred_element_type=jnp.float32)
        or_r[...] = jnp.dot(h, wr_r[...], precision=_HIGH,
                            preferred_element_type=jnp.float32)
        oi_r[...] = inv

    return pl.pallas_call(
        body,
        grid=(N // _BR,),
        in_specs=[pl.BlockSpec((1, _BR, H), lambda i: (0, i, 0)),
                  pl.BlockSpec((1, _BR, H), lambda i: (1, i, 0)),
                  pl.BlockSpec((_BR, 1), lambda i: (i, 0)),
                  pl.BlockSpec((_BR, 1), lambda i: (i, 0)),
                  pl.BlockSpec((_BR, H), lambda i: (i, 0)),
                  pl.BlockSpec((1, H), lambda i: (0, 0)),
                  pl.BlockSpec((H, Hn), lambda i: (0, 0)),
                  pl.BlockSpec((H, Hn), lambda i: (0, 0))],
        out_specs=[pl.BlockSpec((_BR, Hn), lambda i: (i, 0)),
                   pl.BlockSpec((_BR, Hn), lambda i: (i, 0)),
                   pl.BlockSpec((_BR, 1), lambda i: (i, 0))],
        out_shape=[jax.ShapeDtypeStruct((N, Hn), jnp.float32),
                   jax.ShapeDtypeStruct((N, Hn), jnp.float32),
                   jax.ShapeDtypeStruct((N, 1), jnp.float32)],
    )(agg, agg, d0, d1, xr, b, Wl, Wr)


def _combine2(agg, inv, xr, b, Wl, Wr):
    H, Hn = Wl.shape

    def body(a0_r, a1_r, inv_r, xr_r, b_r, wl_r, wr_r, ol_r, or_r):
        h = _layer_out(a0_r[0], a1_r[0], inv_r[...], xr_r[...], b_r[...])
        h = jnp.maximum(h, 0.0)
        ol_r[...] = jnp.dot(h, wl_r[...], precision=_HIGH,
                            preferred_element_type=jnp.float32)
        or_r[...] = jnp.dot(h, wr_r[...], precision=_HIGH,
                            preferred_element_type=jnp.float32)

    return pl.pallas_call(
        body,
        grid=(N // _BR,),
        in_specs=[pl.BlockSpec((1, _BR, H), lambda i: (0, i, 0)),
                  pl.BlockSpec((1, _BR, H), lambda i: (1, i, 0)),
                  pl.BlockSpec((_BR, 1), lambda i: (i, 0)),
                  pl.BlockSpec((_BR, H), lambda i: (i, 0)),
                  pl.BlockSpec((1, H), lambda i: (0, 0)),
                  pl.BlockSpec((H, Hn), lambda i: (0, 0)),
                  pl.BlockSpec((H, Hn), lambda i: (0, 0))],
        out_specs=[pl.BlockSpec((_BR, Hn), lambda i: (i, 0)),
                   pl.BlockSpec((_BR, Hn), lambda i: (i, 0))],
        out_shape=[jax.ShapeDtypeStruct((N, Hn), jnp.float32)] * 2,
    )(agg, agg, inv, xr, b, Wl, Wr)


def _final(agg, inv, xr, b):
    H = xr.shape[1]

    def body(a0_r, a1_r, inv_r, xr_r, b_r, o_r):
        h = _layer_out(a0_r[0], a1_r[0], inv_r[...], xr_r[...], b_r[...])
        m = jnp.max(h, axis=1, keepdims=True)
        e = h - m
        o_r[...] = e - jnp.log(jnp.sum(jnp.exp(e), axis=1, keepdims=True))

    return pl.pallas_call(
        body,
        grid=(N // _BR,),
        in_specs=[pl.BlockSpec((1, _BR, H), lambda i: (0, i, 0)),
                  pl.BlockSpec((1, _BR, H), lambda i: (1, i, 0)),
                  pl.BlockSpec((_BR, 1), lambda i: (i, 0)),
                  pl.BlockSpec((_BR, H), lambda i: (i, 0)),
                  pl.BlockSpec((1, H), lambda i: (0, 0))],
        out_specs=[pl.BlockSpec((_BR, H), lambda i: (i, 0))],
        out_shape=[jax.ShapeDtypeStruct((N, H), jnp.float32)],
    )(agg, agg, inv, xr, b)[0]


def kernel(x, edge_index, W1l, b1, W1r, W2l, b2, W2r, W3l, b3, W3r):
    src2 = edge_index[0].reshape(E // CH, CH)
    dst2 = edge_index[1].reshape(E // CH, CH)

    x1l, x1r = _mm2(x, W1l, W1r)
    agg1, deg0, deg1 = _sc_agg64(x1l, src2, dst2)
    x2l, x2r, inv = _combine1(agg1, deg0[:N].reshape(N, 1),
                              deg1[:N].reshape(N, 1), x1r,
                              b1.reshape(1, H1), W2l, W2r)

    agg2, = _sc_agg32(x2l, src2, dst2)
    x3l, x3r = _combine2(agg2, inv, x2r, b2.reshape(1, H2), W3l, W3r)

    agg3, = _sc_agg32(x3l, src2, dst2)
    return _final(agg3, inv, x3r, b3.reshape(1, OUT))


# speedup vs baseline: 1.1593x; 1.1593x over previous
import functools

import jax
import jax.numpy as jnp
from jax import lax
from jax.experimental import pallas as pl
from jax.experimental.pallas import tpu as pltpu
from jax.experimental.pallas import tpu_sc as plsc

N = 10000
E = 320000
D = 128
H1 = 64
H2 = 32
OUT = 32

CH = 125
NW = 32
ROWS_PER_TILE = E // (NW * CH)
NPAD = 10240
STRIPE = NPAD // 16

_HIGH = jax.lax.Precision.HIGHEST


def _make_sc_agg(H, with_deg):
    mesh = plsc.VectorSubcoreMesh(core_axis_name="c", subcore_axis_name="s")
    out_type = [jax.ShapeDtypeStruct((2, NPAD, H), jnp.float32)]
    if with_deg:
        out_type.append(jax.ShapeDtypeStruct((NPAD,), jnp.float32))
        out_type.append(jax.ShapeDtypeStruct((NPAD,), jnp.float32))
    scratch = [
        pltpu.VMEM((ROWS_PER_TILE, CH), jnp.int32),
        pltpu.VMEM((ROWS_PER_TILE, CH), jnp.int32),
    ]
    nb = 5 if H > 32 else 8
    scratch += [pltpu.VMEM((CH, H), jnp.float32)] * nb
    scratch += [
        pltpu.VMEM((128, H), jnp.float32),
        pltpu.VMEM_SHARED((NPAD, H), jnp.float32),
    ] + [pltpu.SemaphoreType.DMA] * nb
    if with_deg:
        scratch += [
            pltpu.VMEM((CH,), jnp.float32),
            pltpu.VMEM((STRIPE,), jnp.float32),
            pltpu.VMEM_SHARED((NPAD,), jnp.float32),
        ]

    @functools.partial(pl.kernel, mesh=mesh, out_type=tuple(out_type),
                       scratch_types=scratch,
                       compiler_params=pltpu.CompilerParams(
                           use_tc_tiling_on_sc=False))
    def k(xl, src2, dst2, *refs):
        if with_deg:
            (agg_out, deg0_out, deg1_out, src_v, dst_v, *rest) = refs
        else:
            (agg_out, src_v, dst_v, *rest) = refs
        nb = 5 if H > 32 else 8
        bufs = tuple(rest[0:nb])
        zblk = rest[nb]
        acc = rest[nb + 1]
        sems = tuple(rest[nb + 2:2 * nb + 2])
        if with_deg:
            ones_v, zvec, dacc = rest[2 * nb + 2:2 * nb + 5]
        c = lax.axis_index("c")
        s = lax.axis_index("s")
        wid = s * 2 + c
        zero16 = jnp.zeros((16,), jnp.float32)

        def zfill(i, _):
            zblk[i // (H // 16), pl.ds((i % (H // 16)) * 16, 16)] = zero16
            return 0
        lax.fori_loop(0, 128 * (H // 16), zfill, 0)

        def zacc(t, _):
            pltpu.sync_copy(zblk, acc.at[pl.ds(s * STRIPE + t * 128, 128), :])
            return 0
        lax.fori_loop(0, STRIPE // 128, zacc, 0)

        if with_deg:
            def zdfill(i, _):
                zvec[pl.ds(i * 16, 16)] = zero16
                return 0
            lax.fori_loop(0, STRIPE // 16, zdfill, 0)
            pltpu.sync_copy(zvec, dacc.at[pl.ds(s * STRIPE, STRIPE)])
            one16 = jnp.ones((16,), jnp.float32)

            def ofill(i, _):
                ones_v[pl.ds(jnp.minimum(i * 16, CH - 16), 16)] = one16
                return 0
            lax.fori_loop(0, (CH + 15) // 16, ofill, 0)

        plsc.subcore_barrier()

        row0 = wid * ROWS_PER_TILE
        pltpu.sync_copy(src2.at[pl.ds(row0, ROWS_PER_TILE), :], src_v)
        pltpu.sync_copy(dst2.at[pl.ds(row0, ROWS_PER_TILE), :], dst_v)

        NB = nb

        def drain_g(b, sm):
            pltpu.make_async_copy(xl.at[pl.ds(0, CH), :], b, sm).wait()

        for b in range(NB):
            pltpu.async_copy(xl.at[src_v.at[b]], bufs[b], sems[b])

        def step(i, _):
            for b in range(NB):
                j = NB * i + b
                drain_g(bufs[b], sems[b])
                pltpu.sync_copy(bufs[b], acc.at[dst_v.at[j]], add=True)
                if with_deg:
                    pltpu.sync_copy(ones_v, dacc.at[dst_v.at[j]], add=True)
                jn = jnp.minimum(j + NB, ROWS_PER_TILE - 1)
                pltpu.async_copy(xl.at[src_v.at[jn]], bufs[b], sems[b])
            return 0
        lax.fori_loop(0, ROWS_PER_TILE // NB, step, 0)
        for b in range(NB):
            drain_g(bufs[b], sems[b])

        plsc.subcore_barrier()

        pltpu.sync_copy(acc.at[pl.ds(s * STRIPE, STRIPE), :],
                        agg_out.at[c].at[pl.ds(s * STRIPE, STRIPE), :])
        if with_deg:
            @pl.when(c == 0)
            def _():
                pltpu.sync_copy(dacc.at[pl.ds(s * STRIPE, STRIPE)],
                                deg0_out.at[pl.ds(s * STRIPE, STRIPE)])

            @pl.when(c == 1)
            def _():
                pltpu.sync_copy(dacc.at[pl.ds(s * STRIPE, STRIPE)],
                                deg1_out.at[pl.ds(s * STRIPE, STRIPE)])

    return k


_sc_agg64 = _make_sc_agg(H1, with_deg=True)
_sc_agg32 = _make_sc_agg(H2, with_deg=False)


_BR = 2000


def _mm2(x, Wl, Wr):
    Din, H = Wl.shape

    def body(x_ref, wl_ref, wr_ref, ol_ref, or_ref):
        xb = x_ref[...]
        ol_ref[...] = jnp.dot(xb, wl_ref[...], precision=None,
                              preferred_element_type=jnp.float32)
        or_ref[...] = jnp.dot(xb, wr_ref[...], precision=None,
                              preferred_element_type=jnp.float32)

    return pl.pallas_call(
        body,
        grid=(N // _BR,),
        in_specs=[pl.BlockSpec((_BR, Din), lambda i: (i, 0)),
                  pl.BlockSpec((Din, H), lambda i: (0, 0)),
                  pl.BlockSpec((Din, H), lambda i: (0, 0))],
        out_specs=[pl.BlockSpec((_BR, H), lambda i: (i, 0)),
                   pl.BlockSpec((_BR, H), lambda i: (i, 0))],
        out_shape=[jax.ShapeDtypeStruct((N, H), jnp.float32)] * 2,
    )(x, Wl, Wr)


def _layer_out(a0, a1, inv, xr, b):
    out = (a0 + a1) * inv + b + xr
    nrm = jnp.sqrt(jnp.sum(out * out, axis=1, keepdims=True))
    return out / jnp.maximum(nrm, 1e-12)


def _combine1(agg, d0, d1, xr, b, Wl, Wr):
    H, Hn = Wl.shape

    def body(a0_r, a1_r, d0_r, d1_r, xr_r, b_r, wl_r, wr_r,
             ol_r, or_r, oi_r):
        inv = 1.0 / jnp.maximum(d0_r[...] + d1_r[...], 1.0)
        h = _layer_out(a0_r[0], a1_r[0], inv, xr_r[...], b_r[...])
        h = jnp.maximum(h, 0.0)
        ol_r[...] = jnp.dot(h, wl_r[...], precision=None,
                            preferred_element_type=jnp.float32)
        or_r[...] = jnp.dot(h, wr_r[...], precision=None,
                            preferred_element_type=jnp.float32)
        oi_r[...] = inv

    return pl.pallas_call(
        body,
        grid=(N // _BR,),
        in_specs=[pl.BlockSpec((1, _BR, H), lambda i: (0, i, 0)),
                  pl.BlockSpec((1, _BR, H), lambda i: (1, i, 0)),
                  pl.BlockSpec((_BR, 1), lambda i: (i, 0)),
                  pl.BlockSpec((_BR, 1), lambda i: (i, 0)),
                  pl.BlockSpec((_BR, H), lambda i: (i, 0)),
                  pl.BlockSpec((1, H), lambda i: (0, 0)),
                  pl.BlockSpec((H, Hn), lambda i: (0, 0)),
                  pl.BlockSpec((H, Hn), lambda i: (0, 0))],
        out_specs=[pl.BlockSpec((_BR, Hn), lambda i: (i, 0)),
                   pl.BlockSpec((_BR, Hn), lambda i: (i, 0)),
                   pl.BlockSpec((_BR, 1), lambda i: (i, 0))],
        out_shape=[jax.ShapeDtypeStruct((N, Hn), jnp.float32),
                   jax.ShapeDtypeStruct((N, Hn), jnp.float32),
                   jax.ShapeDtypeStruct((N, 1), jnp.float32)],
    )(agg, agg, d0, d1, xr, b, Wl, Wr)


def _combine2(agg, inv, xr, b, Wl, Wr):
    H, Hn = Wl.shape

    def body(a0_r, a1_r, inv_r, xr_r, b_r, wl_r, wr_r, ol_r, or_r):
        h = _layer_out(a0_r[0], a1_r[0], inv_r[...], xr_r[...], b_r[...])
        h = jnp.maximum(h, 0.0)
        ol_r[...] = jnp.dot(h, wl_r[...], precision=None,
                            preferred_element_type=jnp.float32)
        or_r[...] = jnp.dot(h, wr_r[...], precision=None,
                            preferred_element_type=jnp.float32)

    return pl.pallas_call(
        body,
        grid=(N // _BR,),
        in_specs=[pl.BlockSpec((1, _BR, H), lambda i: (0, i, 0)),
                  pl.BlockSpec((1, _BR, H), lambda i: (1, i, 0)),
                  pl.BlockSpec((_BR, 1), lambda i: (i, 0)),
                  pl.BlockSpec((_BR, H), lambda i: (i, 0)),
                  pl.BlockSpec((1, H), lambda i: (0, 0)),
                  pl.BlockSpec((H, Hn), lambda i: (0, 0)),
                  pl.BlockSpec((H, Hn), lambda i: (0, 0))],
        out_specs=[pl.BlockSpec((_BR, Hn), lambda i: (i, 0)),
                   pl.BlockSpec((_BR, Hn), lambda i: (i, 0))],
        out_shape=[jax.ShapeDtypeStruct((N, Hn), jnp.float32)] * 2,
    )(agg, agg, inv, xr, b, Wl, Wr)


def _final(agg, inv, xr, b):
    H = xr.shape[1]

    def body(a0_r, a1_r, inv_r, xr_r, b_r, o_r):
        h = _layer_out(a0_r[0], a1_r[0], inv_r[...], xr_r[...], b_r[...])
        m = jnp.max(h, axis=1, keepdims=True)
        e = h - m
        o_r[...] = e - jnp.log(jnp.sum(jnp.exp(e), axis=1, keepdims=True))

    return pl.pallas_call(
        body,
        grid=(N // _BR,),
        in_specs=[pl.BlockSpec((1, _BR, H), lambda i: (0, i, 0)),
                  pl.BlockSpec((1, _BR, H), lambda i: (1, i, 0)),
                  pl.BlockSpec((_BR, 1), lambda i: (i, 0)),
                  pl.BlockSpec((_BR, H), lambda i: (i, 0)),
                  pl.BlockSpec((1, H), lambda i: (0, 0))],
        out_specs=[pl.BlockSpec((_BR, H), lambda i: (i, 0))],
        out_shape=[jax.ShapeDtypeStruct((N, H), jnp.float32)],
    )(agg, agg, inv, xr, b)[0]


def kernel(x, edge_index, W1l, b1, W1r, W2l, b2, W2r, W3l, b3, W3r):
    src2 = edge_index[0].reshape(E // CH, CH)
    dst2 = edge_index[1].reshape(E // CH, CH)

    x1l, x1r = _mm2(x, W1l, W1r)
    agg1, deg0, deg1 = _sc_agg64(x1l, src2, dst2)
    x2l, x2r, inv = _combine1(agg1, deg0[:N].reshape(N, 1),
                              deg1[:N].reshape(N, 1), x1r,
                              b1.reshape(1, H1), W2l, W2r)

    agg2, = _sc_agg32(x2l, src2, dst2)
    x3l, x3r = _combine2(agg2, inv, x2r, b2.reshape(1, H2), W3l, W3r)

    agg3, = _sc_agg32(x3l, src2, dst2)
    return _final(agg3, inv, x3r, b3.reshape(1, OUT))
